# Initial kernel scaffold; baseline (speedup 1.0000x reference)
import functools

import jax
import jax.numpy as jnp
from jax import lax
from jax.experimental import pallas as pl
from jax.experimental.pallas import tpu as pltpu
from jax.experimental.pallas import tpu_sc as plsc

_NC = 2
_NS = 16
_L = 16


def _gather_call(x, src, dst, pos, batch):
    E = src.shape[0]
    npos = pos.shape[0]
    nw = _NC * _NS
    epw = E // nw
    C = 80
    nch = epw // C
    G = C // _L
    mesh = plsc.VectorSubcoreMesh(core_axis_name="c", subcore_axis_name="s")

    @functools.partial(
        pl.kernel,
        mesh=mesh,
        out_type=[
            jax.ShapeDtypeStruct((E, 128), jnp.float32),
            jax.ShapeDtypeStruct((E, 128), jnp.float32),
            jax.ShapeDtypeStruct((E, 4), jnp.float32),
        ],
        scratch_types=[
            pltpu.VMEM((npos, 3), jnp.float32),
            pltpu.VMEM((npos,), jnp.int32),
            pltpu.VMEM((C,), jnp.int32),
            pltpu.VMEM((C,), jnp.int32),
            pltpu.VMEM((C, 128), jnp.float32),
            pltpu.VMEM((C, 128), jnp.float32),
            pltpu.VMEM((C, 4), jnp.float32),
            pltpu.SemaphoreType.DMA,
            pltpu.SemaphoreType.DMA,
        ],
    )
    def k(x_hbm, src_hbm, dst_hbm, pos_hbm, batch_hbm,
          xs_out, xd_out, vb_out,
          pos_v, batch_v, si_v, di_v, xs_v, xd_v, vb_v, sem1, sem2):
        wid = lax.axis_index("s") * _NC + lax.axis_index("c")
        base0 = wid * epw
        pltpu.sync_copy(pos_hbm, pos_v)
        pltpu.sync_copy(batch_hbm, batch_v)

        def body(ci, carry):
            base = base0 + ci * C
            pltpu.sync_copy(src_hbm.at[pl.ds(base, C)], si_v)
            pltpu.sync_copy(dst_hbm.at[pl.ds(base, C)], di_v)
            cp1 = pltpu.async_copy(x_hbm.at[si_v], xs_v, sem1)
            cp2 = pltpu.async_copy(x_hbm.at[di_v], xd_v, sem2)
            cp1.wait()
            cp2.wait()
            for g in range(G):
                sl = pl.ds(g * _L, _L)
                s16 = si_v[sl]
                d16 = di_v[sl]
                rows = lax.broadcasted_iota(jnp.int32, (_L,), 0) + g * _L
                for c3 in range(3):
                    col = jnp.full((_L,), c3, jnp.int32)
                    ps = plsc.load_gather(pos_v, [s16, col])
                    pd = plsc.load_gather(pos_v, [d16, col])
                    plsc.store_scatter(vb_v, [rows, col], ps - pd)
                b16 = plsc.load_gather(batch_v, [s16])
                plsc.store_scatter(vb_v, [rows, jnp.full((_L,), 3, jnp.int32)],
                                   b16.astype(jnp.float32))
            pltpu.sync_copy(xs_v, xs_out.at[pl.ds(base, C)])
            pltpu.sync_copy(xd_v, xd_out.at[pl.ds(base, C)])
            pltpu.sync_copy(vb_v, vb_out.at[pl.ds(base, C)])
            return carry

        lax.fori_loop(0, nch, body, 0)

    return k(x, src, dst, pos, batch)


def _dense_call(xs, xd, x, vb, We1, be1, We2, be2, Wf1, bf1, Wf2, bf2):
    E = xs.shape[0]
    B = 2000
    NB = E // B
    off = 10000 // B

    def body(xs_r, xd_r, xe_r, vb_r, We1_r, be1_r, We2_r, be2_r,
             Wf1_r, bf1_r, Wf2_r, bf2_r, F_r, ea_r):
        i = pl.program_id(0)
        a = xs_r[...]
        b = xd_r[...]
        c = xe_r[...]

        def head(W1_r, b1_r, W2_r, b2_r):
            h = (jnp.dot(a, W1_r[0:128, :], preferred_element_type=jnp.float32)
                 + jnp.dot(b, W1_r[128:256, :], preferred_element_type=jnp.float32)
                 + jnp.dot(c, W1_r[256:384, :], preferred_element_type=jnp.float32)
                 + b1_r[...])
            h = jnp.maximum(h, 0.0)
            return jnp.dot(h, W2_r[...], preferred_element_type=jnp.float32) + b2_r[...]

        e = head(We1_r, be1_r, We2_r, be2_r)
        f = head(Wf1_r, bf1_r, Wf2_r, bf2_r)
        vbv = vb_r[...]
        v3 = vbv[:, 0:3]
        nrm = jnp.sqrt(jnp.sum(v3 * v3, axis=1, keepdims=True))
        vhat = v3 / jnp.maximum(nrm, 1e-12)
        force = f * vhat
        F_r[...] = jnp.concatenate(
            [force, e, jnp.zeros((B, 4), jnp.float32)], axis=1)
        gid = lax.broadcasted_iota(jnp.float32, (B, 64), 1)
        oh = (vbv[:, 3:4] == gid).astype(jnp.float32)
        part = lax.dot_general(oh, e, (((0,), (0,)), ((), ())),
                               preferred_element_type=jnp.float32)

        @pl.when(i == 0)
        def _init():
            ea_r[...] = jnp.zeros_like(ea_r)

        ea_r[...] += part

    return pl.pallas_call(
        body,
        grid=(NB,),
        in_specs=[
            pl.BlockSpec((B, 128), lambda i: (i, 0)),
            pl.BlockSpec((B, 128), lambda i: (i, 0)),
            pl.BlockSpec((B, 128), lambda i: (i + off, 0)),
            pl.BlockSpec((B, 4), lambda i: (i, 0)),
            pl.BlockSpec((384, 128), lambda i: (0, 0)),
            pl.BlockSpec((1, 128), lambda i: (0, 0)),
            pl.BlockSpec((128, 1), lambda i: (0, 0)),
            pl.BlockSpec((1, 1), lambda i: (0, 0)),
            pl.BlockSpec((384, 128), lambda i: (0, 0)),
            pl.BlockSpec((1, 128), lambda i: (0, 0)),
            pl.BlockSpec((128, 1), lambda i: (0, 0)),
            pl.BlockSpec((1, 1), lambda i: (0, 0)),
        ],
        out_specs=[
            pl.BlockSpec((B, 8), lambda i: (i, 0)),
            pl.BlockSpec((64, 1), lambda i: (0, 0)),
        ],
        out_shape=[
            jax.ShapeDtypeStruct((E, 8), jnp.float32),
            jax.ShapeDtypeStruct((64, 1), jnp.float32),
        ],
    )(xs, xd, x, vb, We1, be1.reshape(1, 128), We2, be2.reshape(1, 1),
      Wf1, bf1.reshape(1, 128), Wf2, bf2.reshape(1, 1))


def _scatter_call(F, src, zeros_tab):
    E = F.shape[0]
    n = zeros_tab.shape[0]
    nw = _NC * _NS
    epw = E // nw
    C = 80
    nch = epw // C
    rpt = n // _NS
    mesh = plsc.VectorSubcoreMesh(core_axis_name="c", subcore_axis_name="s")

    @functools.partial(
        pl.kernel,
        mesh=mesh,
        out_type=jax.ShapeDtypeStruct((_NC * n, 8), jnp.float32),
        scratch_types=[
            pltpu.VMEM((C,), jnp.int32),
            pltpu.VMEM((C, 8), jnp.float32),
            pltpu.VMEM_SHARED((n, 8), jnp.float32),
        ],
    )
    def k(f_hbm, src_hbm, z_hbm, out_hbm, si_v, f_v, tab_sh):
        cid = lax.axis_index("c")
        sid = lax.axis_index("s")
        wid = sid * _NC + cid
        base0 = wid * epw
        pltpu.sync_copy(z_hbm.at[pl.ds(sid * rpt, rpt)],
                        tab_sh.at[pl.ds(sid * rpt, rpt)])
        plsc.subcore_barrier()

        def body(ci, carry):
            base = base0 + ci * C
            pltpu.sync_copy(src_hbm.at[pl.ds(base, C)], si_v)
            pltpu.sync_copy(f_hbm.at[pl.ds(base, C)], f_v)
            pltpu.sync_copy(f_v, tab_sh.at[si_v], add=True)
            return carry

        lax.fori_loop(0, nch, body, 0)
        plsc.subcore_barrier()
        pltpu.sync_copy(tab_sh.at[pl.ds(sid * rpt, rpt)],
                        out_hbm.at[pl.ds(cid * n + sid * rpt, rpt)])

    return k(F, src, zeros_tab)


def _combine_call(tab2):
    n = tab2.shape[0] // 2

    def body(t_r, o_r):
        o_r[...] = t_r[0:n, :] + t_r[n:2 * n, :]

    return pl.pallas_call(
        body,
        out_shape=jax.ShapeDtypeStruct((n, 8), jnp.float32),
    )(tab2)


def kernel(x, pos, batch, edge_index, We1, be1, We2, be2, Wf1, bf1, Wf2, bf2):
    n = pos.shape[0]
    src = edge_index[0]
    dst = edge_index[1]
    xs, xd, vb = _gather_call(x, src, dst, pos, batch)
    F, energy = _dense_call(xs, xd, x, vb, We1, be1, We2, be2,
                            Wf1, bf1, Wf2, bf2)
    tab2 = _scatter_call(F, src, jnp.zeros((n, 8), jnp.float32))
    comb = _combine_call(tab2)
    forces = comb[:, :3]
    return (energy, forces)


# speedup vs baseline: 6.9355x; 6.9355x over previous
import functools

import jax
import jax.numpy as jnp
from jax import lax
from jax.experimental import pallas as pl
from jax.experimental.pallas import tpu as pltpu
from jax.experimental.pallas import tpu_sc as plsc

_NC = 2
_NS = 16
_L = 16


def _gather_call(x, src, dst, posf, batch):
    E = src.shape[0]
    npos3 = posf.shape[0]
    nw = _NC * _NS
    epw = E // nw
    C = 80
    nch = epw // C
    G = C // _L
    mesh = plsc.VectorSubcoreMesh(core_axis_name="c", subcore_axis_name="s")

    @functools.partial(
        pl.kernel,
        mesh=mesh,
        out_type=[
            jax.ShapeDtypeStruct((E, 128), jnp.float32),
            jax.ShapeDtypeStruct((E, 128), jnp.float32),
            jax.ShapeDtypeStruct((E * 4,), jnp.float32),
        ],
        scratch_types=[
            pltpu.VMEM((npos3,), jnp.float32),
            pltpu.VMEM((npos3 // 3,), jnp.int32),
            pltpu.VMEM((C,), jnp.int32),
            pltpu.VMEM((C,), jnp.int32),
            pltpu.VMEM((C, 128), jnp.float32),
            pltpu.VMEM((C, 128), jnp.float32),
            pltpu.VMEM((C * 4,), jnp.float32),
            pltpu.SemaphoreType.DMA,
            pltpu.SemaphoreType.DMA,
        ],
        compiler_params=pltpu.CompilerParams(needs_layout_passes=False),
    )
    def k(x_hbm, src_hbm, dst_hbm, pos_hbm, batch_hbm,
          xs_out, xd_out, vb_out,
          pos_v, batch_v, si_v, di_v, xs_v, xd_v, vb_v, sem1, sem2):
        wid = lax.axis_index("s") * _NC + lax.axis_index("c")
        base0 = wid * epw
        pltpu.sync_copy(pos_hbm, pos_v)
        pltpu.sync_copy(batch_hbm, batch_v)

        def body(ci, carry):
            base = base0 + ci * C
            pltpu.sync_copy(src_hbm.at[pl.ds(base, C)], si_v)
            pltpu.sync_copy(dst_hbm.at[pl.ds(base, C)], di_v)
            cp1 = pltpu.async_copy(x_hbm.at[si_v], xs_v, sem1)
            cp2 = pltpu.async_copy(x_hbm.at[di_v], xd_v, sem2)
            cp1.wait()
            cp2.wait()
            for g in range(G):
                sl = pl.ds(g * _L, _L)
                s16 = si_v[sl]
                d16 = di_v[sl]
                rows4 = (lax.broadcasted_iota(jnp.int32, (_L,), 0) + g * _L) * 4
                s3 = s16 * 3
                d3 = d16 * 3
                for c3 in range(3):
                    ps = plsc.load_gather(pos_v, [s3 + c3])
                    pd = plsc.load_gather(pos_v, [d3 + c3])
                    plsc.store_scatter(vb_v, [rows4 + c3], ps - pd)
                b16 = plsc.load_gather(batch_v, [s16])
                plsc.store_scatter(vb_v, [rows4 + 3], b16.astype(jnp.float32))
            pltpu.sync_copy(xs_v, xs_out.at[pl.ds(base, C)])
            pltpu.sync_copy(xd_v, xd_out.at[pl.ds(base, C)])
            pltpu.sync_copy(vb_v, vb_out.at[pl.ds(base * 4, C * 4)])
            return carry

        lax.fori_loop(0, nch, body, 0)

    return k(x, src, dst, posf, batch)


def _dense_call(xs, xd, x, vb, We1, be1, We2, be2, Wf1, bf1, Wf2, bf2):
    E = xs.shape[0]
    B = 2000
    NB = E // B
    off = 10000 // B

    def body(xs_r, xd_r, xe_r, vb_r, We1_r, be1_r, We2_r, be2_r,
             Wf1_r, bf1_r, Wf2_r, bf2_r, F_r, ea_r):
        i = pl.program_id(0)
        a = xs_r[...]
        b = xd_r[...]
        c = xe_r[...]

        def head(W1_r, b1_r, W2_r, b2_r):
            h = (jnp.dot(a, W1_r[0:128, :], preferred_element_type=jnp.float32)
                 + jnp.dot(b, W1_r[128:256, :], preferred_element_type=jnp.float32)
                 + jnp.dot(c, W1_r[256:384, :], preferred_element_type=jnp.float32)
                 + b1_r[...])
            h = jnp.maximum(h, 0.0)
            return jnp.dot(h, W2_r[...], preferred_element_type=jnp.float32) + b2_r[...]

        e = head(We1_r, be1_r, We2_r, be2_r)
        f = head(Wf1_r, bf1_r, Wf2_r, bf2_r)
        vbv = vb_r[...]
        v3 = vbv[:, 0:3]
        nrm = jnp.sqrt(jnp.sum(v3 * v3, axis=1, keepdims=True))
        vhat = v3 / jnp.maximum(nrm, 1e-12)
        force = f * vhat
        F_r[...] = jnp.concatenate([force, e], axis=1)
        gid = lax.broadcasted_iota(jnp.int32, (B, 64), 1).astype(jnp.float32)
        oh = (vbv[:, 3:4] == gid).astype(jnp.float32)
        part = lax.dot_general(oh, e, (((0,), (0,)), ((), ())),
                               preferred_element_type=jnp.float32)

        @pl.when(i == 0)
        def _init():
            ea_r[...] = jnp.zeros_like(ea_r)

        ea_r[...] += part

    return pl.pallas_call(
        body,
        grid=(NB,),
        in_specs=[
            pl.BlockSpec((B, 128), lambda i: (i, 0)),
            pl.BlockSpec((B, 128), lambda i: (i, 0)),
            pl.BlockSpec((B, 128), lambda i: (i + off, 0)),
            pl.BlockSpec((B, 4), lambda i: (i, 0)),
            pl.BlockSpec((384, 128), lambda i: (0, 0)),
            pl.BlockSpec((1, 128), lambda i: (0, 0)),
            pl.BlockSpec((128, 1), lambda i: (0, 0)),
            pl.BlockSpec((1, 1), lambda i: (0, 0)),
            pl.BlockSpec((384, 128), lambda i: (0, 0)),
            pl.BlockSpec((1, 128), lambda i: (0, 0)),
            pl.BlockSpec((128, 1), lambda i: (0, 0)),
            pl.BlockSpec((1, 1), lambda i: (0, 0)),
        ],
        out_specs=[
            pl.BlockSpec((B, 4), lambda i: (i, 0)),
            pl.BlockSpec((64, 1), lambda i: (0, 0)),
        ],
        out_shape=[
            jax.ShapeDtypeStruct((E, 4), jnp.float32),
            jax.ShapeDtypeStruct((64, 1), jnp.float32),
        ],
    )(xs, xd, x, vb, We1, be1.reshape(1, 128), We2, be2.reshape(1, 1),
      Wf1, bf1.reshape(1, 128), Wf2, bf2.reshape(1, 1))


def _scatter_call(F, src, zeros_tab):
    E = F.shape[0]
    T3 = zeros_tab.shape[0]
    nw = _NC * _NS
    epw = E // nw
    C = 80
    nch = epw // C
    G = C // _L
    mesh = plsc.VectorSubcoreMesh(core_axis_name="c", subcore_axis_name="s")

    @functools.partial(
        pl.kernel,
        mesh=mesh,
        out_type=jax.ShapeDtypeStruct((nw, T3), jnp.float32),
        scratch_types=[
            pltpu.VMEM((C,), jnp.int32),
            pltpu.VMEM((C, 4), jnp.float32),
            pltpu.VMEM((T3,), jnp.float32),
        ],
        compiler_params=pltpu.CompilerParams(needs_layout_passes=False),
    )
    def k(f_hbm, src_hbm, z_hbm, out_hbm, si_v, f_v, tab_v):
        wid = lax.axis_index("s") * _NC + lax.axis_index("c")
        base0 = wid * epw
        pltpu.sync_copy(z_hbm, tab_v)
        lane = lax.broadcasted_iota(jnp.int32, (_L,), 0)
        masks = [lane == l for l in range(_L)]

        def body(ci, carry):
            base = base0 + ci * C
            pltpu.sync_copy(src_hbm.at[pl.ds(base, C)], si_v)
            pltpu.sync_copy(f_hbm.at[pl.ds(base, C)], f_v)
            for g in range(G):
                rows = lane + g * _L
                s3 = si_v[pl.ds(g * _L, _L)] * 3
                for c3 in range(3):
                    val = plsc.load_gather(f_v, [rows, jnp.full((_L,), c3, jnp.int32)])
                    tgt = s3 + c3
                    for m in masks:
                        plsc.addupdate_scatter(tab_v, [tgt], val, mask=m)
            return carry

        lax.fori_loop(0, nch, body, 0)
        pltpu.sync_copy(tab_v, out_hbm.at[wid])

    return k(F, src, zeros_tab)


def _combine_call(tabs):
    nw, T3 = tabs.shape

    def body(t_r, o_r):
        o_r[...] = jnp.sum(t_r[...], axis=0, keepdims=True)

    return pl.pallas_call(
        body,
        out_shape=jax.ShapeDtypeStruct((1, T3), jnp.float32),
    )(tabs)


def kernel(x, pos, batch, edge_index, We1, be1, We2, be2, Wf1, bf1, Wf2, bf2):
    n = pos.shape[0]
    src = edge_index[0]
    dst = edge_index[1]
    xs, xd, vbf = _gather_call(x, src, dst, pos.reshape(-1), batch)
    vb = vbf.reshape(-1, 4)
    F, energy = _dense_call(xs, xd, x, vb, We1, be1, We2, be2,
                            Wf1, bf1, Wf2, bf2)
    npad = ((n + 127) // 128) * 128
    tabs = _scatter_call(F, src, jnp.zeros((3 * npad,), jnp.float32))
    comb = _combine_call(tabs)
    forces = comb.reshape(npad, 3)[:n]
    return (energy, forces)


# speedup vs baseline: 7.0568x; 1.0175x over previous
import functools

import jax
import jax.numpy as jnp
from jax import lax
from jax.experimental import pallas as pl
from jax.experimental.pallas import tpu as pltpu
from jax.experimental.pallas import tpu_sc as plsc

_NC = 2
_NS = 16
_L = 16


def _gather_call(pa, pb, src, dst, posf, batch):
    E = src.shape[0]
    npos3 = posf.shape[0]
    nw = _NC * _NS
    epw = E // nw
    C = 80
    nch = epw // C
    G = C // _L
    mesh = plsc.VectorSubcoreMesh(core_axis_name="c", subcore_axis_name="s")

    @functools.partial(
        pl.kernel,
        mesh=mesh,
        out_type=[
            jax.ShapeDtypeStruct((E, 128), jnp.float32),
            jax.ShapeDtypeStruct((E, 128), jnp.float32),
            jax.ShapeDtypeStruct((E * 4,), jnp.float32),
        ],
        scratch_types=[
            pltpu.VMEM((npos3,), jnp.float32),
            pltpu.VMEM((npos3 // 3,), jnp.int32),
            pltpu.VMEM((C,), jnp.int32),
            pltpu.VMEM((C,), jnp.int32),
            pltpu.VMEM((C, 128), jnp.float32),
            pltpu.VMEM((C, 128), jnp.float32),
            pltpu.VMEM((C * 4,), jnp.float32),
            pltpu.SemaphoreType.DMA,
            pltpu.SemaphoreType.DMA,
        ],
        compiler_params=pltpu.CompilerParams(needs_layout_passes=False),
    )
    def k(pa_hbm, pb_hbm, src_hbm, dst_hbm, pos_hbm, batch_hbm,
          xs_out, xd_out, vb_out,
          pos_v, batch_v, si_v, di_v, xs_v, xd_v, vb_v, sem1, sem2):
        wid = lax.axis_index("s") * _NC + lax.axis_index("c")
        base0 = wid * epw
        pltpu.sync_copy(pos_hbm, pos_v)
        pltpu.sync_copy(batch_hbm, batch_v)

        def body(ci, carry):
            base = base0 + ci * C
            pltpu.sync_copy(src_hbm.at[pl.ds(base, C)], si_v)
            pltpu.sync_copy(dst_hbm.at[pl.ds(base, C)], di_v)
            cp1 = pltpu.async_copy(pa_hbm.at[si_v], xs_v, sem1)
            cp2 = pltpu.async_copy(pb_hbm.at[di_v], xd_v, sem2)
            cp1.wait()
            cp2.wait()
            for g in range(G):
                sl = pl.ds(g * _L, _L)
                s16 = si_v[sl]
                d16 = di_v[sl]
                rows4 = (lax.broadcasted_iota(jnp.int32, (_L,), 0) + g * _L) * 4
                s3 = s16 * 3
                d3 = d16 * 3
                for c3 in range(3):
                    ps = plsc.load_gather(pos_v, [s3 + c3])
                    pd = plsc.load_gather(pos_v, [d3 + c3])
                    plsc.store_scatter(vb_v, [rows4 + c3], ps - pd)
                b16 = plsc.load_gather(batch_v, [s16])
                plsc.store_scatter(vb_v, [rows4 + 3], b16.astype(jnp.float32))
            pltpu.sync_copy(xs_v, xs_out.at[pl.ds(base, C)])
            pltpu.sync_copy(xd_v, xd_out.at[pl.ds(base, C)])
            pltpu.sync_copy(vb_v, vb_out.at[pl.ds(base * 4, C * 4)])
            return carry

        lax.fori_loop(0, nch, body, 0)

    return k(pa, pb, src, dst, posf, batch)


def _project_call(xn, Wcat):
    n = xn.shape[0]
    B = 2000
    NB = n // B

    def body(x_r, w_r, o_r):
        o_r[...] = jnp.dot(x_r[...].astype(jnp.bfloat16), w_r[...],
                           preferred_element_type=jnp.float32
                           ).astype(jnp.bfloat16)

    return pl.pallas_call(
        body,
        grid=(NB,),
        in_specs=[
            pl.BlockSpec((B, 128), lambda i: (i, 0)),
            pl.BlockSpec((128, 512), lambda i: (0, 0)),
        ],
        out_specs=pl.BlockSpec((B, 512), lambda i: (i, 0)),
        out_shape=jax.ShapeDtypeStruct((n, 512), jnp.bfloat16),
    )(xn, Wcat)


def _dense_call(gs, gd, x, vb, Wc, be1, We2, be2, bf1, Wf2, bf2):
    E = gs.shape[0]
    B = 2000
    NB = E // B
    off = 10000 // B

    def body(gs_r, gd_r, xe_r, vb_r, Wc_r, be1_r, We2_r, be2_r,
             bf1_r, Wf2_r, bf2_r, F_r, ea_r):
        i = pl.program_id(0)

        def unpack(ref):
            u = lax.bitcast_convert_type(ref[...], jnp.int32)
            lo = lax.bitcast_convert_type(u << 16, jnp.float32)
            hi = lax.bitcast_convert_type(u & jnp.int32(-65536), jnp.float32)
            return lo, hi

        s_e, s_f = unpack(gs_r)
        d_e, d_f = unpack(gd_r)
        c = xe_r[...].astype(jnp.bfloat16)
        q = jnp.dot(c, Wc_r[...], preferred_element_type=jnp.float32)
        he = jnp.maximum(s_e + d_e + q[:, 0:128] + be1_r[...], 0.0)
        hf = jnp.maximum(s_f + d_f + q[:, 128:256] + bf1_r[...], 0.0)
        e = jnp.dot(he, We2_r[...], preferred_element_type=jnp.float32) + be2_r[...]
        f = jnp.dot(hf, Wf2_r[...], preferred_element_type=jnp.float32) + bf2_r[...]
        vbv = vb_r[...]
        v3 = vbv[:, 0:3]
        nrm = jnp.sqrt(jnp.sum(v3 * v3, axis=1, keepdims=True))
        vhat = v3 / jnp.maximum(nrm, 1e-12)
        force = f * vhat
        F_r[...] = jnp.concatenate([force, e], axis=1)
        gid = lax.broadcasted_iota(jnp.int32, (B, 64), 1).astype(jnp.float32)
        oh = (vbv[:, 3:4] == gid).astype(jnp.float32)
        part = lax.dot_general(oh, e, (((0,), (0,)), ((), ())),
                               preferred_element_type=jnp.float32)

        @pl.when(i == 0)
        def _init():
            ea_r[...] = jnp.zeros_like(ea_r)

        ea_r[...] += part

    return pl.pallas_call(
        body,
        grid=(NB,),
        in_specs=[
            pl.BlockSpec((B, 128), lambda i: (i, 0)),
            pl.BlockSpec((B, 128), lambda i: (i, 0)),
            pl.BlockSpec((B, 128), lambda i: (i + off, 0)),
            pl.BlockSpec((B, 4), lambda i: (i, 0)),
            pl.BlockSpec((128, 256), lambda i: (0, 0)),
            pl.BlockSpec((1, 128), lambda i: (0, 0)),
            pl.BlockSpec((128, 1), lambda i: (0, 0)),
            pl.BlockSpec((1, 1), lambda i: (0, 0)),
            pl.BlockSpec((1, 128), lambda i: (0, 0)),
            pl.BlockSpec((128, 1), lambda i: (0, 0)),
            pl.BlockSpec((1, 1), lambda i: (0, 0)),
        ],
        out_specs=[
            pl.BlockSpec((B, 4), lambda i: (i, 0)),
            pl.BlockSpec((64, 1), lambda i: (0, 0)),
        ],
        out_shape=[
            jax.ShapeDtypeStruct((E, 4), jnp.float32),
            jax.ShapeDtypeStruct((64, 1), jnp.float32),
        ],
    )(gs, gd, x, vb,
      Wc, be1.reshape(1, 128), We2, be2.reshape(1, 1),
      bf1.reshape(1, 128), Wf2, bf2.reshape(1, 1))


def _scatter_call(F, src, zeros_tab):
    E = F.shape[0]
    T3 = zeros_tab.shape[0]
    nw = _NC * _NS
    epw = E // nw
    C = 80
    nch = epw // C
    G = C // _L
    mesh = plsc.VectorSubcoreMesh(core_axis_name="c", subcore_axis_name="s")

    @functools.partial(
        pl.kernel,
        mesh=mesh,
        out_type=jax.ShapeDtypeStruct((nw, T3), jnp.float32),
        scratch_types=[
            pltpu.VMEM((C,), jnp.int32),
            pltpu.VMEM((C, 4), jnp.float32),
            pltpu.VMEM((T3,), jnp.float32),
        ],
        compiler_params=pltpu.CompilerParams(needs_layout_passes=False),
    )
    def k(f_hbm, src_hbm, z_hbm, out_hbm, si_v, f_v, tab_v):
        wid = lax.axis_index("s") * _NC + lax.axis_index("c")
        base0 = wid * epw
        pltpu.sync_copy(z_hbm, tab_v)
        lane = lax.broadcasted_iota(jnp.int32, (_L,), 0)
        masks = [lane == l for l in range(_L)]

        def body(ci, carry):
            base = base0 + ci * C
            pltpu.sync_copy(src_hbm.at[pl.ds(base, C)], si_v)
            pltpu.sync_copy(f_hbm.at[pl.ds(base, C)], f_v)
            for g in range(G):
                rows = lane + g * _L
                s3 = si_v[pl.ds(g * _L, _L)] * 3
                for c3 in range(3):
                    val = plsc.load_gather(f_v, [rows, jnp.full((_L,), c3, jnp.int32)])
                    tgt = s3 + c3
                    for m in masks:
                        plsc.addupdate_scatter(tab_v, [tgt], val, mask=m)
            return carry

        lax.fori_loop(0, nch, body, 0)
        pltpu.sync_copy(tab_v, out_hbm.at[wid])

    return k(F, src, zeros_tab)


def _combine_call(tabs):
    nw, T3 = tabs.shape

    def body(t_r, o_r):
        o_r[...] = jnp.sum(t_r[...], axis=0, keepdims=True)

    return pl.pallas_call(
        body,
        out_shape=jax.ShapeDtypeStruct((1, T3), jnp.float32),
    )(tabs)


def kernel(x, pos, batch, edge_index, We1, be1, We2, be2, Wf1, bf1, Wf2, bf2):
    n = pos.shape[0]
    src = edge_index[0]
    dst = edge_index[1]
    Wcat = jnp.concatenate(
        [We1[0:128], Wf1[0:128], We1[128:256], Wf1[128:256]],
        axis=1).astype(jnp.bfloat16)
    P = _project_call(x[:n], Wcat)
    pa = lax.bitcast_convert_type(
        jnp.stack([P[:, 0:128], P[:, 128:256]], axis=-1), jnp.float32)
    pb = lax.bitcast_convert_type(
        jnp.stack([P[:, 256:384], P[:, 384:512]], axis=-1), jnp.float32)
    gs, gd, vbf = _gather_call(pa, pb, src, dst, pos.reshape(-1), batch)
    vb = vbf.reshape(-1, 4)
    Wc = jnp.concatenate([We1[256:384], Wf1[256:384]],
                         axis=1).astype(jnp.bfloat16)
    F, energy = _dense_call(gs, gd, x, vb, Wc, be1, We2, be2, bf1, Wf2, bf2)
    npad = ((n + 127) // 128) * 128
    tabs = _scatter_call(F, src, jnp.zeros((3 * npad,), jnp.float32))
    comb = _combine_call(tabs)
    forces = comb.reshape(npad, 3)[:n]
    return (energy, forces)


# speedup vs baseline: 7.5477x; 1.0696x over previous
import functools

import jax
import jax.numpy as jnp
from jax import lax
from jax.experimental import pallas as pl
from jax.experimental.pallas import tpu as pltpu
from jax.experimental.pallas import tpu_sc as plsc

_NC = 2
_NS = 16
_L = 16


def _gather_call(pa, pb, src, dst, posf, batch):
    E = src.shape[0]
    npos3 = posf.shape[0]
    nw = _NC * _NS
    epw = E // nw
    C = 80
    nch = epw // C
    G = C // _L
    mesh = plsc.VectorSubcoreMesh(core_axis_name="c", subcore_axis_name="s")

    @functools.partial(
        pl.kernel,
        mesh=mesh,
        out_type=[
            jax.ShapeDtypeStruct((E, 128), jnp.float32),
            jax.ShapeDtypeStruct((E, 128), jnp.float32),
            jax.ShapeDtypeStruct((E * 4,), jnp.float32),
        ],
        scratch_types=[
            pltpu.VMEM((npos3,), jnp.float32),
            pltpu.VMEM((npos3 // 3,), jnp.int32),
            pltpu.VMEM((C,), jnp.int32),
            pltpu.VMEM((C,), jnp.int32),
            pltpu.VMEM((C, 128), jnp.float32),
            pltpu.VMEM((C, 128), jnp.float32),
            pltpu.VMEM((C,), jnp.int32),
            pltpu.VMEM((C,), jnp.int32),
            pltpu.VMEM((C, 128), jnp.float32),
            pltpu.VMEM((C, 128), jnp.float32),
            pltpu.VMEM((C * 4,), jnp.float32),
            pltpu.SemaphoreType.DMA,
            pltpu.SemaphoreType.DMA,
            pltpu.SemaphoreType.DMA,
            pltpu.SemaphoreType.DMA,
        ],
        compiler_params=pltpu.CompilerParams(needs_layout_passes=False),
    )
    def k(pa_hbm, pb_hbm, src_hbm, dst_hbm, pos_hbm, batch_hbm,
          xs_out, xd_out, vb_out,
          pos_v, batch_v, si0, di0, xs0, xd0, si1, di1, xs1, xd1, vb_v,
          sa0, sb0, sa1, sb1):
        wid = lax.axis_index("s") * _NC + lax.axis_index("c")
        base0 = wid * epw
        pltpu.sync_copy(pos_hbm, pos_v)
        pltpu.sync_copy(batch_hbm, batch_v)
        slots = ((si0, di0, xs0, xd0, sa0, sb0), (si1, di1, xs1, xd1, sa1, sb1))

        def issue(ci, slot):
            si_v, di_v, xs_v, xd_v, sa, sb = slots[slot]
            base = base0 + ci * C
            pltpu.sync_copy(src_hbm.at[pl.ds(base, C)], si_v)
            pltpu.sync_copy(dst_hbm.at[pl.ds(base, C)], di_v)
            pltpu.async_copy(pa_hbm.at[si_v], xs_v, sa)
            pltpu.async_copy(pb_hbm.at[di_v], xd_v, sb)

        def visit(ci, slot):
            si_v, di_v, xs_v, xd_v, sa, sb = slots[slot]
            pltpu.make_async_copy(pa_hbm.at[si_v], xs_v, sa).wait()
            pltpu.make_async_copy(pb_hbm.at[di_v], xd_v, sb).wait()
            for g in range(G):
                sl = pl.ds(g * _L, _L)
                s16 = si_v[sl]
                d16 = di_v[sl]
                rows4 = (lax.broadcasted_iota(jnp.int32, (_L,), 0) + g * _L) * 4
                s3 = s16 * 3
                d3 = d16 * 3
                for c3 in range(3):
                    ps = plsc.load_gather(pos_v, [s3 + c3])
                    pd = plsc.load_gather(pos_v, [d3 + c3])
                    plsc.store_scatter(vb_v, [rows4 + c3], ps - pd)
                b16 = plsc.load_gather(batch_v, [s16])
                plsc.store_scatter(vb_v, [rows4 + 3], b16.astype(jnp.float32))

            @pl.when(ci + 1 < nch)
            def _():
                issue(ci + 1, 1 - slot)

            base = base0 + ci * C
            pltpu.sync_copy(xs_v, xs_out.at[pl.ds(base, C)])
            pltpu.sync_copy(xd_v, xd_out.at[pl.ds(base, C)])
            pltpu.sync_copy(vb_v, vb_out.at[pl.ds(base * 4, C * 4)])

        issue(0, 0)

        def body(j, carry):
            visit(2 * j, 0)
            visit(2 * j + 1, 1)
            return carry

        lax.fori_loop(0, nch // 2, body, 0)
        if nch % 2:
            visit(nch - 1, 0)

    return k(pa, pb, src, dst, posf, batch)


def _project_call(xn, Wcat):
    n = xn.shape[0]
    B = 2000
    NB = n // B

    def body(x_r, w_r, o_r):
        o_r[...] = jnp.dot(x_r[...].astype(jnp.bfloat16), w_r[...],
                           preferred_element_type=jnp.float32
                           ).astype(jnp.bfloat16)

    return pl.pallas_call(
        body,
        grid=(NB,),
        in_specs=[
            pl.BlockSpec((B, 128), lambda i: (i, 0)),
            pl.BlockSpec((128, 512), lambda i: (0, 0)),
        ],
        out_specs=pl.BlockSpec((B, 512), lambda i: (i, 0)),
        out_shape=jax.ShapeDtypeStruct((n, 512), jnp.bfloat16),
    )(xn, Wcat)


def _dense_call(gs, gd, x, vb, Wc, be1, We2, be2, bf1, Wf2, bf2):
    E = gs.shape[0]
    B = 2000
    NB = E // B
    off = 10000 // B

    def body(gs_r, gd_r, xe_r, vb_r, Wc_r, be1_r, We2_r, be2_r,
             bf1_r, Wf2_r, bf2_r, F_r, ea_r):
        i = pl.program_id(0)

        def unpack(ref):
            u = lax.bitcast_convert_type(ref[...], jnp.int32)
            lo = lax.bitcast_convert_type(u << 16, jnp.float32)
            hi = lax.bitcast_convert_type(u & jnp.int32(-65536), jnp.float32)
            return lo, hi

        s_e, s_f = unpack(gs_r)
        d_e, d_f = unpack(gd_r)
        c = xe_r[...].astype(jnp.bfloat16)
        q = jnp.dot(c, Wc_r[...], preferred_element_type=jnp.float32)
        he = jnp.maximum(s_e + d_e + q[:, 0:128] + be1_r[...], 0.0)
        hf = jnp.maximum(s_f + d_f + q[:, 128:256] + bf1_r[...], 0.0)
        e = jnp.dot(he, We2_r[...], preferred_element_type=jnp.float32) + be2_r[...]
        f = jnp.dot(hf, Wf2_r[...], preferred_element_type=jnp.float32) + bf2_r[...]
        vbv = vb_r[...]
        v3 = vbv[:, 0:3]
        nrm = jnp.sqrt(jnp.sum(v3 * v3, axis=1, keepdims=True))
        vhat = v3 / jnp.maximum(nrm, 1e-12)
        force = f * vhat
        F_r[...] = jnp.concatenate([force, e], axis=1)
        gid = lax.broadcasted_iota(jnp.int32, (B, 64), 1).astype(jnp.float32)
        oh = (vbv[:, 3:4] == gid).astype(jnp.float32)
        part = lax.dot_general(oh, e, (((0,), (0,)), ((), ())),
                               preferred_element_type=jnp.float32)

        @pl.when(i == 0)
        def _init():
            ea_r[...] = jnp.zeros_like(ea_r)

        ea_r[...] += part

    return pl.pallas_call(
        body,
        grid=(NB,),
        in_specs=[
            pl.BlockSpec((B, 128), lambda i: (i, 0)),
            pl.BlockSpec((B, 128), lambda i: (i, 0)),
            pl.BlockSpec((B, 128), lambda i: (i + off, 0)),
            pl.BlockSpec((B, 4), lambda i: (i, 0)),
            pl.BlockSpec((128, 256), lambda i: (0, 0)),
            pl.BlockSpec((1, 128), lambda i: (0, 0)),
            pl.BlockSpec((128, 1), lambda i: (0, 0)),
            pl.BlockSpec((1, 1), lambda i: (0, 0)),
            pl.BlockSpec((1, 128), lambda i: (0, 0)),
            pl.BlockSpec((128, 1), lambda i: (0, 0)),
            pl.BlockSpec((1, 1), lambda i: (0, 0)),
        ],
        out_specs=[
            pl.BlockSpec((B, 4), lambda i: (i, 0)),
            pl.BlockSpec((64, 1), lambda i: (0, 0)),
        ],
        out_shape=[
            jax.ShapeDtypeStruct((E, 4), jnp.float32),
            jax.ShapeDtypeStruct((64, 1), jnp.float32),
        ],
    )(gs, gd, x, vb,
      Wc, be1.reshape(1, 128), We2, be2.reshape(1, 1),
      bf1.reshape(1, 128), Wf2, bf2.reshape(1, 1))


def _scatter_call(F, src, zeros_tab):
    E = F.shape[0]
    T3 = zeros_tab.shape[0]
    nw = _NC * _NS
    epw = E // nw
    C = 80
    nch = epw // C
    G = C // _L
    mesh = plsc.VectorSubcoreMesh(core_axis_name="c", subcore_axis_name="s")

    @functools.partial(
        pl.kernel,
        mesh=mesh,
        out_type=jax.ShapeDtypeStruct((nw, T3), jnp.float32),
        scratch_types=[
            pltpu.VMEM((C,), jnp.int32),
            pltpu.VMEM((C, 4), jnp.float32),
            pltpu.VMEM((T3,), jnp.float32),
        ],
        compiler_params=pltpu.CompilerParams(needs_layout_passes=False),
    )
    def k(f_hbm, src_hbm, z_hbm, out_hbm, si_v, f_v, tab_v):
        wid = lax.axis_index("s") * _NC + lax.axis_index("c")
        base0 = wid * epw
        pltpu.sync_copy(z_hbm, tab_v)
        lane = lax.broadcasted_iota(jnp.int32, (_L,), 0)
        masks = [lane == l for l in range(_L)]

        def body(ci, carry):
            base = base0 + ci * C
            pltpu.sync_copy(src_hbm.at[pl.ds(base, C)], si_v)
            pltpu.sync_copy(f_hbm.at[pl.ds(base, C)], f_v)
            for g in range(G):
                rows = lane + g * _L
                s16 = si_v[pl.ds(g * _L, _L)]
                s3 = s16 * 3
                srt = lax.sort(s16)
                nxt = lax.gather(
                    srt, jnp.minimum(lane + 1, _L - 1)[:, None],
                    lax.GatherDimensionNumbers(
                        offset_dims=(), collapsed_slice_dims=(0,),
                        start_index_map=(0,)),
                    (1,), mode=lax.GatherScatterMode.PROMISE_IN_BOUNDS)
                dup = jnp.any(jnp.logical_and(srt == nxt, lane < _L - 1))
                vals = [plsc.load_gather(
                    f_v, [rows, jnp.full((_L,), c3, jnp.int32)])
                    for c3 in range(3)]

                @pl.when(jnp.logical_not(dup))
                def _fast():
                    for c3 in range(3):
                        plsc.addupdate_scatter(tab_v, [s3 + c3], vals[c3])

                @pl.when(dup)
                def _slow():
                    for c3 in range(3):
                        for m in masks:
                            plsc.addupdate_scatter(tab_v, [s3 + c3],
                                                   vals[c3], mask=m)
            return carry

        lax.fori_loop(0, nch, body, 0)
        pltpu.sync_copy(tab_v, out_hbm.at[wid])

    return k(F, src, zeros_tab)


def _combine_call(tabs):
    nw, T3 = tabs.shape

    def body(t_r, o_r):
        o_r[...] = jnp.sum(t_r[...], axis=0, keepdims=True)

    return pl.pallas_call(
        body,
        out_shape=jax.ShapeDtypeStruct((1, T3), jnp.float32),
    )(tabs)


def kernel(x, pos, batch, edge_index, We1, be1, We2, be2, Wf1, bf1, Wf2, bf2):
    n = pos.shape[0]
    src = edge_index[0]
    dst = edge_index[1]
    Wcat = jnp.concatenate(
        [We1[0:128], Wf1[0:128], We1[128:256], Wf1[128:256]],
        axis=1).astype(jnp.bfloat16)
    P = _project_call(x[:n], Wcat)
    pa = lax.bitcast_convert_type(
        jnp.stack([P[:, 0:128], P[:, 128:256]], axis=-1), jnp.float32)
    pb = lax.bitcast_convert_type(
        jnp.stack([P[:, 256:384], P[:, 384:512]], axis=-1), jnp.float32)
    gs, gd, vbf = _gather_call(pa, pb, src, dst, pos.reshape(-1), batch)
    vb = vbf.reshape(-1, 4)
    Wc = jnp.concatenate([We1[256:384], Wf1[256:384]],
                         axis=1).astype(jnp.bfloat16)
    F, energy = _dense_call(gs, gd, x, vb, Wc, be1, We2, be2, bf1, Wf2, bf2)
    npad = ((n + 127) // 128) * 128
    tabs = _scatter_call(F, src, jnp.zeros((3 * npad,), jnp.float32))
    comb = _combine_call(tabs)
    forces = comb.reshape(npad, 3)[:n]
    return (energy, forces)


# speedup vs baseline: 8.1522x; 1.0801x over previous
import functools

import jax
import jax.numpy as jnp
from jax import lax
from jax.experimental import pallas as pl
from jax.experimental.pallas import tpu as pltpu
from jax.experimental.pallas import tpu_sc as plsc

_NC = 2
_NS = 16
_L = 16


def _gather_call(pa, pb, src, dst, posf, batch):
    E = src.shape[0]
    npos3 = posf.shape[0]
    nw = _NC * _NS
    epw = E // nw
    C = 80
    nch = epw // C
    G = C // _L
    mesh = plsc.VectorSubcoreMesh(core_axis_name="c", subcore_axis_name="s")

    @functools.partial(
        pl.kernel,
        mesh=mesh,
        out_type=[
            jax.ShapeDtypeStruct((E, 128), jnp.float32),
            jax.ShapeDtypeStruct((E, 128), jnp.float32),
            jax.ShapeDtypeStruct((E * 4,), jnp.float32),
        ],
        scratch_types=[
            pltpu.VMEM((npos3,), jnp.float32),
            pltpu.VMEM((npos3 // 3,), jnp.int32),
            pltpu.VMEM((C,), jnp.int32),
            pltpu.VMEM((C,), jnp.int32),
            pltpu.VMEM((C, 128), jnp.float32),
            pltpu.VMEM((C, 128), jnp.float32),
            pltpu.VMEM((C,), jnp.int32),
            pltpu.VMEM((C,), jnp.int32),
            pltpu.VMEM((C, 128), jnp.float32),
            pltpu.VMEM((C, 128), jnp.float32),
            pltpu.VMEM((C * 4,), jnp.float32),
            pltpu.SemaphoreType.DMA,
            pltpu.SemaphoreType.DMA,
            pltpu.SemaphoreType.DMA,
            pltpu.SemaphoreType.DMA,
        ],
        compiler_params=pltpu.CompilerParams(needs_layout_passes=False),
    )
    def k(pa_hbm, pb_hbm, src_hbm, dst_hbm, pos_hbm, batch_hbm,
          xs_out, xd_out, vb_out,
          pos_v, batch_v, si0, di0, xs0, xd0, si1, di1, xs1, xd1, vb_v,
          sa0, sb0, sa1, sb1):
        wid = lax.axis_index("s") * _NC + lax.axis_index("c")
        base0 = wid * epw
        pltpu.sync_copy(pos_hbm, pos_v)
        pltpu.sync_copy(batch_hbm, batch_v)
        slots = ((si0, di0, xs0, xd0, sa0, sb0), (si1, di1, xs1, xd1, sa1, sb1))

        def issue(ci, slot):
            si_v, di_v, xs_v, xd_v, sa, sb = slots[slot]
            base = base0 + ci * C
            pltpu.sync_copy(src_hbm.at[pl.ds(base, C)], si_v)
            pltpu.sync_copy(dst_hbm.at[pl.ds(base, C)], di_v)
            pltpu.async_copy(pa_hbm.at[si_v], xs_v, sa)
            pltpu.async_copy(pb_hbm.at[di_v], xd_v, sb)

        def visit(ci, slot):
            si_v, di_v, xs_v, xd_v, sa, sb = slots[slot]
            pltpu.make_async_copy(pa_hbm.at[si_v], xs_v, sa).wait()
            pltpu.make_async_copy(pb_hbm.at[di_v], xd_v, sb).wait()
            for g in range(G):
                sl = pl.ds(g * _L, _L)
                s16 = si_v[sl]
                d16 = di_v[sl]
                rows4 = (lax.broadcasted_iota(jnp.int32, (_L,), 0) + g * _L) * 4
                s3 = s16 * 3
                d3 = d16 * 3
                for c3 in range(3):
                    ps = plsc.load_gather(pos_v, [s3 + c3])
                    pd = plsc.load_gather(pos_v, [d3 + c3])
                    plsc.store_scatter(vb_v, [rows4 + c3], ps - pd)
                b16 = plsc.load_gather(batch_v, [s16])
                plsc.store_scatter(vb_v, [rows4 + 3], b16.astype(jnp.float32))

            @pl.when(ci + 1 < nch)
            def _():
                issue(ci + 1, 1 - slot)

            base = base0 + ci * C
            pltpu.sync_copy(xs_v, xs_out.at[pl.ds(base, C)])
            pltpu.sync_copy(xd_v, xd_out.at[pl.ds(base, C)])
            pltpu.sync_copy(vb_v, vb_out.at[pl.ds(base * 4, C * 4)])

        issue(0, 0)

        def body(j, carry):
            visit(2 * j, 0)
            visit(2 * j + 1, 1)
            return carry

        lax.fori_loop(0, nch // 2, body, 0)
        if nch % 2:
            visit(nch - 1, 0)

    return k(pa, pb, src, dst, posf, batch)


def _project_call(xn, Wcat):
    n = xn.shape[0]
    B = 2000
    NB = n // B

    def body(x_r, w_r, o_r):
        o_r[...] = jnp.dot(x_r[...].astype(jnp.bfloat16), w_r[...],
                           preferred_element_type=jnp.float32
                           ).astype(jnp.bfloat16)

    return pl.pallas_call(
        body,
        grid=(NB,),
        in_specs=[
            pl.BlockSpec((B, 128), lambda i: (i, 0)),
            pl.BlockSpec((128, 512), lambda i: (0, 0)),
        ],
        out_specs=pl.BlockSpec((B, 512), lambda i: (i, 0)),
        out_shape=jax.ShapeDtypeStruct((n, 512), jnp.bfloat16),
    )(xn, Wcat)


def _dense_call(gs, gd, x, vb, Wc, be1, We2, be2, bf1, Wf2, bf2):
    E = gs.shape[0]
    B = 2000
    NB = E // B
    off = 10000 // B

    def body(gs_r, gd_r, xe_r, vb_r, Wc_r, be1_r, We2_r, be2_r,
             bf1_r, Wf2_r, bf2_r, F_r, ea_r):
        i = pl.program_id(0)

        def unpack(ref):
            u = lax.bitcast_convert_type(ref[...], jnp.int32)
            lo = lax.bitcast_convert_type(u << 16, jnp.float32)
            hi = lax.bitcast_convert_type(u & jnp.int32(-65536), jnp.float32)
            return lo, hi

        s_e, s_f = unpack(gs_r)
        d_e, d_f = unpack(gd_r)
        c = xe_r[...].astype(jnp.bfloat16)
        q = jnp.dot(c, Wc_r[...], preferred_element_type=jnp.float32)
        he = jnp.maximum(s_e + d_e + q[:, 0:128] + be1_r[...], 0.0)
        hf = jnp.maximum(s_f + d_f + q[:, 128:256] + bf1_r[...], 0.0)
        e = jnp.dot(he, We2_r[...], preferred_element_type=jnp.float32) + be2_r[...]
        f = jnp.dot(hf, Wf2_r[...], preferred_element_type=jnp.float32) + bf2_r[...]
        vbv = vb_r[...]
        v3 = vbv[:, 0:3]
        nrm = jnp.sqrt(jnp.sum(v3 * v3, axis=1, keepdims=True))
        vhat = v3 / jnp.maximum(nrm, 1e-12)
        force = f * vhat
        F_r[...] = jnp.concatenate([force, e], axis=1)
        gid = lax.broadcasted_iota(jnp.int32, (B, 64), 1).astype(jnp.float32)
        oh = (vbv[:, 3:4] == gid).astype(jnp.float32)
        part = lax.dot_general(oh, e, (((0,), (0,)), ((), ())),
                               preferred_element_type=jnp.float32)

        @pl.when(i == 0)
        def _init():
            ea_r[...] = jnp.zeros_like(ea_r)

        ea_r[...] += part

    return pl.pallas_call(
        body,
        grid=(NB,),
        in_specs=[
            pl.BlockSpec((B, 128), lambda i: (i, 0)),
            pl.BlockSpec((B, 128), lambda i: (i, 0)),
            pl.BlockSpec((B, 128), lambda i: (i + off, 0)),
            pl.BlockSpec((B, 4), lambda i: (i, 0)),
            pl.BlockSpec((128, 256), lambda i: (0, 0)),
            pl.BlockSpec((1, 128), lambda i: (0, 0)),
            pl.BlockSpec((128, 1), lambda i: (0, 0)),
            pl.BlockSpec((1, 1), lambda i: (0, 0)),
            pl.BlockSpec((1, 128), lambda i: (0, 0)),
            pl.BlockSpec((128, 1), lambda i: (0, 0)),
            pl.BlockSpec((1, 1), lambda i: (0, 0)),
        ],
        out_specs=[
            pl.BlockSpec((B, 4), lambda i: (i, 0)),
            pl.BlockSpec((64, 1), lambda i: (0, 0)),
        ],
        out_shape=[
            jax.ShapeDtypeStruct((E, 4), jnp.float32),
            jax.ShapeDtypeStruct((64, 1), jnp.float32),
        ],
    )(gs, gd, x, vb,
      Wc, be1.reshape(1, 128), We2, be2.reshape(1, 1),
      bf1.reshape(1, 128), Wf2, bf2.reshape(1, 1))


def _scatter_call(F, src, zeros_tab):
    E = F.shape[0]
    T3 = zeros_tab.shape[0]
    nw = _NC * _NS
    epw = E // nw
    C = 400
    nch = epw // C
    G = C // _L
    mesh = plsc.VectorSubcoreMesh(core_axis_name="c", subcore_axis_name="s")

    @functools.partial(
        pl.kernel,
        mesh=mesh,
        out_type=jax.ShapeDtypeStruct((nw, T3), jnp.float32),
        scratch_types=[
            pltpu.VMEM((C,), jnp.int32),
            pltpu.VMEM((C, 4), jnp.float32),
            pltpu.VMEM((T3,), jnp.float32),
        ],
        compiler_params=pltpu.CompilerParams(needs_layout_passes=False),
    )
    def k(f_hbm, src_hbm, z_hbm, out_hbm, si_v, f_v, tab_v):
        wid = lax.axis_index("s") * _NC + lax.axis_index("c")
        base0 = wid * epw
        pltpu.sync_copy(z_hbm, tab_v)
        lane = lax.broadcasted_iota(jnp.int32, (_L,), 0)
        masks = [lane == l for l in range(_L)]

        def body(ci, carry):
            base = base0 + ci * C
            pltpu.sync_copy(src_hbm.at[pl.ds(base, C)], si_v)
            pltpu.sync_copy(f_hbm.at[pl.ds(base, C)], f_v)
            for g in range(G):
                rows = lane + g * _L
                s16 = si_v[pl.ds(g * _L, _L)]
                s3 = s16 * 3
                srt = lax.sort(s16)
                nxt = lax.gather(
                    srt, jnp.minimum(lane + 1, _L - 1)[:, None],
                    lax.GatherDimensionNumbers(
                        offset_dims=(), collapsed_slice_dims=(0,),
                        start_index_map=(0,)),
                    (1,), mode=lax.GatherScatterMode.PROMISE_IN_BOUNDS)
                dup = jnp.any(jnp.logical_and(srt == nxt, lane < _L - 1))
                vals = [plsc.load_gather(
                    f_v, [rows, jnp.full((_L,), c3, jnp.int32)])
                    for c3 in range(3)]

                @pl.when(jnp.logical_not(dup))
                def _fast():
                    for c3 in range(3):
                        plsc.addupdate_scatter(tab_v, [s3 + c3], vals[c3])

                @pl.when(dup)
                def _slow():
                    for c3 in range(3):
                        for m in masks:
                            plsc.addupdate_scatter(tab_v, [s3 + c3],
                                                   vals[c3], mask=m)
            return carry

        lax.fori_loop(0, nch, body, 0)
        pltpu.sync_copy(tab_v, out_hbm.at[wid])

    return k(F, src, zeros_tab)


def _combine_call(tabs):
    nw, T3 = tabs.shape

    def body(t_r, o_r):
        o_r[...] = jnp.sum(t_r[...], axis=0, keepdims=True)

    return pl.pallas_call(
        body,
        out_shape=jax.ShapeDtypeStruct((1, T3), jnp.float32),
    )(tabs)


def kernel(x, pos, batch, edge_index, We1, be1, We2, be2, Wf1, bf1, Wf2, bf2):
    n = pos.shape[0]
    src = edge_index[0]
    dst = edge_index[1]
    Wcat = jnp.concatenate(
        [We1[0:128], Wf1[0:128], We1[128:256], Wf1[128:256]],
        axis=1).astype(jnp.bfloat16)
    P = _project_call(x[:n], Wcat)
    pa = lax.bitcast_convert_type(
        jnp.stack([P[:, 0:128], P[:, 128:256]], axis=-1), jnp.float32)
    pb = lax.bitcast_convert_type(
        jnp.stack([P[:, 256:384], P[:, 384:512]], axis=-1), jnp.float32)
    gs, gd, vbf = _gather_call(pa, pb, src, dst, pos.reshape(-1), batch)
    vb = vbf.reshape(-1, 4)
    Wc = jnp.concatenate([We1[256:384], Wf1[256:384]],
                         axis=1).astype(jnp.bfloat16)
    F, energy = _dense_call(gs, gd, x, vb, Wc, be1, We2, be2, bf1, Wf2, bf2)
    npad = ((n + 127) // 128) * 128
    tabs = _scatter_call(F, src, jnp.zeros((3 * npad,), jnp.float32))
    comb = _combine_call(tabs)
    forces = comb.reshape(npad, 3)[:n]
    return (energy, forces)
